# baseline (speedup 1.0000x reference)
import functools

import jax
import jax.numpy as jnp
from jax import lax
from jax.experimental import pallas as pl
from jax.experimental.pallas import tpu as pltpu
from jax.experimental.pallas import tpu_sc as plsc

N_CORES = 2
N_SUBCORES = 16
NW = N_CORES * N_SUBCORES

BATCH = 16384
EMB = 64
HID = 768

B_PER_W = BATCH // NW
CHUNK = 128
N_CHUNKS = B_PER_W // CHUNK


def _gather_body(heads, rels, tails, ent_tab, rel_tab,
                 out_h, out_r, out_t, idx_v, rows_v, sem):
    wid = lax.axis_index("s") * N_CORES + lax.axis_index("c")
    base = wid * B_PER_W
    for idx_hbm, table, out_hbm in ((heads, ent_tab, out_h),
                                    (rels, rel_tab, out_r),
                                    (tails, ent_tab, out_t)):
        for c in range(N_CHUNKS):
            off = base + c * CHUNK
            pltpu.sync_copy(idx_hbm.at[pl.ds(off, CHUNK)], idx_v)
            pltpu.async_copy(table.at[idx_v], rows_v, sem).wait()
            pltpu.sync_copy(rows_v, out_hbm.at[pl.ds(off, CHUNK)])


_gather = pl.kernel(
    _gather_body,
    out_type=(jax.ShapeDtypeStruct((BATCH, EMB), jnp.float32),) * 3,
    mesh=plsc.VectorSubcoreMesh(core_axis_name="c", subcore_axis_name="s"),
    scratch_types=[
        pltpu.VMEM((CHUNK,), jnp.int32),
        pltpu.VMEM((CHUNK, EMB), jnp.float32),
        pltpu.SemaphoreType.DMA,
    ],
    compiler_params=pltpu.CompilerParams(use_tc_tiling_on_sc=False),
)


MM_BLK = 1024


def _mm_body(h_ref, r_ref, t_ref, w_ref, b_ref, oh_ref, or_ref, ot_ref):
    w = w_ref[...]
    bias = b_ref[...]
    oh_ref[...] = jnp.dot(h_ref[...], w, preferred_element_type=jnp.float32) + bias
    or_ref[...] = jnp.dot(r_ref[...], w, preferred_element_type=jnp.float32) + bias
    ot_ref[...] = jnp.dot(t_ref[...], w, preferred_element_type=jnp.float32) + bias


def _project(h_rows, r_rows, t_rows, W, b2):
    row_spec = pl.BlockSpec((MM_BLK, EMB), lambda i: (i, 0))
    out_spec = pl.BlockSpec((MM_BLK, HID), lambda i: (i, 0))
    return pl.pallas_call(
        _mm_body,
        grid=(BATCH // MM_BLK,),
        in_specs=[
            row_spec, row_spec, row_spec,
            pl.BlockSpec((EMB, HID), lambda i: (0, 0)),
            pl.BlockSpec((1, HID), lambda i: (0, 0)),
        ],
        out_specs=[out_spec, out_spec, out_spec],
        out_shape=(jax.ShapeDtypeStruct((BATCH, HID), jnp.float32),) * 3,
    )(h_rows, r_rows, t_rows, W, b2)


@jax.jit
def kernel(triples, ent_emb, rel_emb, W, b):
    heads = triples[:, 0]
    rels = triples[:, 1]
    tails = triples[:, 2]
    h_rows, r_rows, t_rows = _gather(heads, rels, tails, ent_emb, rel_emb)
    return _project(h_rows, r_rows, t_rows, W, b.reshape(1, HID))
